# baseline (speedup 1.0000x reference)
import functools

import jax
import jax.numpy as jnp
from jax import lax
from jax.experimental import pallas as pl
from jax.experimental.pallas import tpu as pltpu
from jax.experimental.pallas import tpu_sc as plsc

N = 65536
D = 512
H = 1024

NW = 32
RPW = N // NW
GCH = 128
LOGGCH = 7
HALF = N // 2
BM = 1024
LOGBM = 10
NSTEPS = N // BM

_INV_SQRT2 = 0.7071067811865476

_MESH = plsc.VectorSubcoreMesh(core_axis_name="c", subcore_axis_name="s")


@functools.partial(
    pl.kernel,
    out_type=[
        jax.ShapeDtypeStruct((NW, RPW), jnp.int32),
        jax.ShapeDtypeStruct((NW, 16), jnp.int32),
    ],
    mesh=_MESH,
    compiler_params=pltpu.CompilerParams(needs_layout_passes=False),
    scratch_types=[
        pltpu.VMEM((RPW,), jnp.int32),
        pltpu.VMEM((RPW,), jnp.int32),
        pltpu.VMEM((16,), jnp.int32),
        pltpu.VMEM((16,), jnp.int32),
    ],
)
def _compact(ids_hbm, task_hbm, idx_hbm, cnts_hbm,
             ids_v, idx_v, task_v, cnt_v):
    cid = lax.axis_index("c")
    sid = lax.axis_index("s")
    wid = sid * 2 + cid
    base = wid * RPW
    pltpu.sync_copy(ids_hbm.at[pl.ds(base, RPW)], ids_v)
    pltpu.sync_copy(task_hbm, task_v)
    t = task_v[...]
    lanes = lax.iota(jnp.int32, 16)
    zeros16 = jnp.zeros((16,), jnp.int32)

    def _zero(i, c):
        idx_v[pl.ds(i * 16, 16)] = zeros16
        return c

    lax.fori_loop(0, RPW // 16, _zero, 0)

    def _step(i, ofs):
        v = ids_v[pl.ds(i * 16, 16)]
        m = v == t
        rows = (base + i * 16) + lanes
        cs = plsc.cumsum(jnp.where(m, zeros16 + 1, zeros16))
        pos = ofs + cs - 1
        plsc.store_scatter(idx_v, [pos], rows, mask=m)
        return ofs + jnp.sum(jnp.where(m, zeros16 + 1, zeros16), axis=0)

    ofs = lax.fori_loop(0, RPW // 16, _step, jnp.zeros((16,), jnp.int32))
    cnt_v[...] = ofs
    pltpu.sync_copy(cnt_v, cnts_hbm.at[wid])
    pltpu.sync_copy(idx_v, idx_hbm.at[wid])


@functools.partial(
    pl.kernel,
    out_type=jax.ShapeDtypeStruct((N, D), jnp.float32),
    mesh=_MESH,
    compiler_params=pltpu.CompilerParams(needs_layout_passes=False),
    scratch_types=[
        pltpu.VMEM((RPW,), jnp.int32),
        pltpu.VMEM((16,), jnp.int32),
        pltpu.VMEM((16,), jnp.int32),
        pltpu.VMEM((GCH, D), jnp.float32),
        pltpu.SemaphoreType.DMA,
        pltpu.SemaphoreType.DMA,
    ],
)
def _gather(x0_hbm, idx_hbm, cnts_hbm, offs_hbm, xg_hbm,
            idx_v, cnt_v, off_v, rows_v, gsem, wsem):
    cid = lax.axis_index("c")
    sid = lax.axis_index("s")
    wid = sid * 2 + cid
    pltpu.sync_copy(idx_hbm.at[wid], idx_v)
    pltpu.sync_copy(cnts_hbm.at[wid], cnt_v)
    pltpu.sync_copy(offs_hbm.at[wid], off_v)
    cnt = jnp.max(cnt_v[...], axis=0)
    off = jnp.max(off_v[...], axis=0)
    nch = (cnt + GCH - 1) >> LOGGCH
    cbase = pl.multiple_of(cid * HALF + off, 8)

    def _chunk(c, carry):
        pltpu.async_copy(x0_hbm.at[idx_v.at[pl.ds(c * GCH, GCH)]], rows_v,
                         gsem).wait()
        pltpu.sync_copy(rows_v, xg_hbm.at[pl.ds(cbase + c * GCH, GCH)])
        return carry

    lax.fori_loop(0, nch, _chunk, 0)


def _mlp_body(s_ref, w1_ref, b1_ref, w2_ref, b2_ref, x_any, hc_ref,
              xb, sem):
    i = pl.program_id(0)
    na0 = (s_ref[0] + BM - 1) >> LOGBM
    na1 = (s_ref[1] + BM - 1) >> LOGBM
    na = na0 + na1

    def base_of(j):
        return pl.multiple_of(
            jnp.where(j < na0, j << LOGBM, HALF + ((j - na0) << LOGBM)), BM)

    def start(j, p):
        pltpu.make_async_copy(x_any.at[pl.ds(base_of(j), BM), :],
                              xb.at[p], sem.at[p]).start()

    @pl.when(i == 0)
    def _():
        @pl.when(na > 0)
        def _():
            start(0, 0)

    @pl.when(i < na)
    def _():
        @pl.when(i + 1 < na)
        def _():
            start(i + 1, (i + 1) % 2)
        p = i % 2
        pltpu.make_async_copy(x_any.at[pl.ds(base_of(i), BM), :],
                              xb.at[p], sem.at[p]).wait()
        x = xb[p]
        h = jnp.dot(x, w1_ref[...], preferred_element_type=jnp.float32)
        h = h + b1_ref[...]
        g = 0.5 * h * (1.0 + jax.lax.erf(h * _INV_SQRT2))
        o = jnp.sum(g * w2_ref[...], axis=1) + b2_ref[0]
        hc_ref[pl.ds(base_of(i), BM)] = o


def _mlp_stream(tot2, xg, W1, b1r, w2r, b2f):
    grid_spec = pltpu.PrefetchScalarGridSpec(
        num_scalar_prefetch=1,
        grid=(NSTEPS,),
        in_specs=[
            pl.BlockSpec((D, H), lambda i, s: (0, 0)),
            pl.BlockSpec((1, H), lambda i, s: (0, 0)),
            pl.BlockSpec((1, H), lambda i, s: (0, 0)),
            pl.BlockSpec(memory_space=pltpu.SMEM),
            pl.BlockSpec(memory_space=pl.ANY),
        ],
        out_specs=pl.BlockSpec((N,), lambda i, s: (0,)),
        scratch_shapes=[
            pltpu.VMEM((2, BM, D), jnp.float32),
            pltpu.SemaphoreType.DMA((2,)),
        ],
    )
    return pl.pallas_call(
        _mlp_body,
        grid_spec=grid_spec,
        out_shape=jax.ShapeDtypeStruct((N,), jnp.float32),
    )(tot2, W1, b1r, w2r, b2f, xg)


@functools.partial(
    pl.kernel,
    out_type=jax.ShapeDtypeStruct((N,), jnp.float32),
    mesh=_MESH,
    compiler_params=pltpu.CompilerParams(needs_layout_passes=False),
    scratch_types=[
        pltpu.VMEM((RPW,), jnp.int32),
        pltpu.VMEM((RPW,), jnp.float32),
        pltpu.VMEM((RPW,), jnp.float32),
        pltpu.VMEM((16,), jnp.int32),
        pltpu.VMEM((16,), jnp.int32),
    ],
)
def _scatter_back(idx_hbm, cnts_hbm, offs_hbm, hc_hbm, ob_hbm, out_hbm,
                  idx_v, hv_v, ob_v, cnt_v, off_v):
    cid = lax.axis_index("c")
    sid = lax.axis_index("s")
    wid = sid * 2 + cid
    base = wid * RPW
    pltpu.sync_copy(cnts_hbm.at[wid], cnt_v)
    pltpu.sync_copy(offs_hbm.at[wid], off_v)
    pltpu.sync_copy(ob_hbm.at[pl.ds(base, RPW)], ob_v)
    pltpu.sync_copy(idx_hbm.at[wid], idx_v)
    cnt = jnp.max(cnt_v[...], axis=0)
    off = jnp.max(off_v[...], axis=0)
    hoff = pl.multiple_of(cid * HALF + off, 8)
    pltpu.sync_copy(hc_hbm.at[pl.ds(hoff, RPW)], hv_v)
    lanes = lax.iota(jnp.int32, 16)

    def _scatter(j, carry):
        pos = idx_v[pl.ds(j * 16, 16)] - base
        vals = hv_v[pl.ds(j * 16, 16)]
        valid = (j * 16 + lanes) < cnt
        plsc.store_scatter(ob_v, [pos], vals, mask=valid)
        return carry

    lax.fori_loop(0, (cnt + 15) >> 4, _scatter, 0)
    pltpu.sync_copy(ob_v, out_hbm.at[pl.ds(base, RPW)])


def kernel(x0, task_ids, out_buf, task, W1, b1, W2, b2):
    ids = task_ids.reshape(N).astype(jnp.int32)
    taskv = jnp.full((16,), task, jnp.int32)
    idxm, counts = _compact(ids, taskv)
    padded = (((counts[:, 0] + GCH - 1) // GCH) * GCH).reshape(16, 2)
    excl = jnp.cumsum(padded, axis=0) - padded
    offs = jnp.broadcast_to(excl.reshape(NW, 1), (NW, 16)).astype(jnp.int32)
    tot2 = padded.sum(axis=0).astype(jnp.int32)
    xg = _gather(x0, idxm, counts, offs)
    hc = _mlp_stream(tot2, xg, W1, b1.reshape(1, H), W2.reshape(1, H),
                     b2.reshape(1))
    out = _scatter_back(idxm, counts, offs, hc, out_buf.reshape(N))
    return out.reshape(N, 1)


# speedup vs baseline: 1.3449x; 1.3449x over previous
import functools

import jax
import jax.numpy as jnp
from jax import lax
from jax.experimental import pallas as pl
from jax.experimental.pallas import tpu as pltpu
from jax.experimental.pallas import tpu_sc as plsc

N = 65536
D = 512
H = 1024

NW = 32
RPW = N // NW
GCH = 64
LOGGCH = 6
HALF = N // 2
BM = 1024
LOGBM = 10
NSTEPS = N // BM

_INV_SQRT2 = 0.7071067811865476

_MESH = plsc.VectorSubcoreMesh(core_axis_name="c", subcore_axis_name="s")


@functools.partial(
    pl.kernel,
    out_type=[
        jax.ShapeDtypeStruct((NW, RPW), jnp.int32),
        jax.ShapeDtypeStruct((NW, 16), jnp.int32),
    ],
    mesh=_MESH,
    compiler_params=pltpu.CompilerParams(needs_layout_passes=False),
    scratch_types=[
        pltpu.VMEM((RPW,), jnp.int32),
        pltpu.VMEM((RPW,), jnp.int32),
        pltpu.VMEM((16,), jnp.int32),
        pltpu.VMEM((16,), jnp.int32),
    ],
)
def _compact(ids_hbm, task_hbm, idx_hbm, cnts_hbm,
             ids_v, idx_v, task_v, cnt_v):
    cid = lax.axis_index("c")
    sid = lax.axis_index("s")
    wid = sid * 2 + cid
    base = wid * RPW
    pltpu.sync_copy(ids_hbm.at[pl.ds(base, RPW)], ids_v)
    pltpu.sync_copy(task_hbm, task_v)
    t = task_v[...]
    lanes = lax.iota(jnp.int32, 16)
    zeros16 = jnp.zeros((16,), jnp.int32)

    def _zero(i, c):
        idx_v[pl.ds(i * 16, 16)] = zeros16
        return c

    lax.fori_loop(0, RPW // 16, _zero, 0)

    def _step(i, ofs):
        v = ids_v[pl.ds(i * 16, 16)]
        m = v == t
        rows = (base + i * 16) + lanes
        cs = plsc.cumsum(jnp.where(m, zeros16 + 1, zeros16))
        pos = ofs + cs - 1
        plsc.store_scatter(idx_v, [pos], rows, mask=m)
        return ofs + jnp.sum(jnp.where(m, zeros16 + 1, zeros16), axis=0)

    ofs = lax.fori_loop(0, RPW // 16, _step, jnp.zeros((16,), jnp.int32))
    cnt_v[...] = ofs
    pltpu.sync_copy(cnt_v, cnts_hbm.at[wid])
    pltpu.sync_copy(idx_v, idx_hbm.at[wid])


@functools.partial(
    pl.kernel,
    out_type=jax.ShapeDtypeStruct((N, D), jnp.float32),
    mesh=_MESH,
    compiler_params=pltpu.CompilerParams(needs_layout_passes=False),
    scratch_types=[
        pltpu.VMEM((RPW,), jnp.int32),
        pltpu.VMEM((16,), jnp.int32),
        pltpu.VMEM((16,), jnp.int32),
        pltpu.VMEM((2, GCH, D), jnp.float32),
        pltpu.SemaphoreType.DMA((2,)),
        pltpu.SemaphoreType.DMA((2,)),
    ],
)
def _gather(x0_hbm, idx_hbm, cnts_hbm, offs_hbm, xg_hbm,
            idx_v, cnt_v, off_v, rows_v, gsem, wsem):
    cid = lax.axis_index("c")
    sid = lax.axis_index("s")
    wid = sid * 2 + cid
    pltpu.sync_copy(idx_hbm.at[wid], idx_v)
    pltpu.sync_copy(cnts_hbm.at[wid], cnt_v)
    pltpu.sync_copy(offs_hbm.at[wid], off_v)
    cnt = jnp.max(cnt_v[...], axis=0)
    off = jnp.max(off_v[...], axis=0)
    nch = (cnt + GCH - 1) >> LOGGCH
    cbase = pl.multiple_of(cid * HALF + off, 8)

    def _g(c, p):
        return pltpu.make_async_copy(
            x0_hbm.at[idx_v.at[pl.ds(c * GCH, GCH)]], rows_v.at[p],
            gsem.at[p])

    def _w(c, p):
        return pltpu.make_async_copy(
            rows_v.at[p], xg_hbm.at[pl.ds(cbase + c * GCH, GCH)],
            wsem.at[p])

    @pl.when(nch > 0)
    def _():
        _g(0, 0).start()

    def _chunk(c, carry):
        p = c & 1
        _g(c, p).wait()

        @pl.when(c >= 1)
        def _():
            _w(c - 1, 1 - p).wait()
        _w(c, p).start()

        @pl.when(c + 1 < nch)
        def _():
            _g(c + 1, 1 - p).start()
        return carry

    lax.fori_loop(0, nch, _chunk, 0)

    @pl.when(nch > 0)
    def _():
        _w(nch - 1, (nch - 1) & 1).wait()


def _mlp_body(s_ref, w1_ref, b1_ref, w2_ref, b2_ref, x_any, hc_ref,
              xb, sem):
    i = pl.program_id(0)
    na0 = (s_ref[0] + BM - 1) >> LOGBM
    na1 = (s_ref[1] + BM - 1) >> LOGBM
    na = na0 + na1

    def base_of(j):
        return pl.multiple_of(
            jnp.where(j < na0, j << LOGBM, HALF + ((j - na0) << LOGBM)), BM)

    def start(j, p):
        pltpu.make_async_copy(x_any.at[pl.ds(base_of(j), BM), :],
                              xb.at[p], sem.at[p]).start()

    @pl.when(i == 0)
    def _():
        @pl.when(na > 0)
        def _():
            start(0, 0)

        @pl.when(na > 1)
        def _():
            start(1, 1)

        @pl.when(na > 2)
        def _():
            start(2, 2)

    @pl.when(i < na)
    def _():
        @pl.when(i + 3 < na)
        def _():
            start(i + 3, (i + 3) % 4)
        p = i % 4
        pltpu.make_async_copy(x_any.at[pl.ds(base_of(i), BM), :],
                              xb.at[p], sem.at[p]).wait()
        x = xb[p]
        h = jnp.dot(x, w1_ref[...], preferred_element_type=jnp.float32)
        h = h + b1_ref[...]
        g = 0.5 * h * (1.0 + jax.lax.erf(h * _INV_SQRT2))
        o = jnp.sum(g * w2_ref[...], axis=1) + b2_ref[0]
        hc_ref[pl.ds(base_of(i), BM)] = o


def _mlp_stream(tot2, xg, W1, b1r, w2r, b2f):
    grid_spec = pltpu.PrefetchScalarGridSpec(
        num_scalar_prefetch=1,
        grid=(NSTEPS,),
        in_specs=[
            pl.BlockSpec((D, H), lambda i, s: (0, 0)),
            pl.BlockSpec((1, H), lambda i, s: (0, 0)),
            pl.BlockSpec((1, H), lambda i, s: (0, 0)),
            pl.BlockSpec(memory_space=pltpu.SMEM),
            pl.BlockSpec(memory_space=pl.ANY),
        ],
        out_specs=pl.BlockSpec((N,), lambda i, s: (0,)),
        scratch_shapes=[
            pltpu.VMEM((4, BM, D), jnp.float32),
            pltpu.SemaphoreType.DMA((4,)),
        ],
    )
    return pl.pallas_call(
        _mlp_body,
        grid_spec=grid_spec,
        out_shape=jax.ShapeDtypeStruct((N,), jnp.float32),
    )(tot2, W1, b1r, w2r, b2f, xg)


@functools.partial(
    pl.kernel,
    out_type=jax.ShapeDtypeStruct((N,), jnp.float32),
    mesh=_MESH,
    compiler_params=pltpu.CompilerParams(needs_layout_passes=False),
    scratch_types=[
        pltpu.VMEM((RPW,), jnp.int32),
        pltpu.VMEM((RPW,), jnp.float32),
        pltpu.VMEM((RPW,), jnp.float32),
        pltpu.VMEM((16,), jnp.int32),
        pltpu.VMEM((16,), jnp.int32),
    ],
)
def _scatter_back(idx_hbm, cnts_hbm, offs_hbm, hc_hbm, ob_hbm, out_hbm,
                  idx_v, hv_v, ob_v, cnt_v, off_v):
    cid = lax.axis_index("c")
    sid = lax.axis_index("s")
    wid = sid * 2 + cid
    base = wid * RPW
    pltpu.sync_copy(cnts_hbm.at[wid], cnt_v)
    pltpu.sync_copy(offs_hbm.at[wid], off_v)
    pltpu.sync_copy(ob_hbm.at[pl.ds(base, RPW)], ob_v)
    pltpu.sync_copy(idx_hbm.at[wid], idx_v)
    cnt = jnp.max(cnt_v[...], axis=0)
    off = jnp.max(off_v[...], axis=0)
    hoff = pl.multiple_of(cid * HALF + off, 8)
    pltpu.sync_copy(hc_hbm.at[pl.ds(hoff, RPW)], hv_v)
    lanes = lax.iota(jnp.int32, 16)

    def _scatter(j, carry):
        pos = idx_v[pl.ds(j * 16, 16)] - base
        vals = hv_v[pl.ds(j * 16, 16)]
        valid = (j * 16 + lanes) < cnt
        plsc.store_scatter(ob_v, [pos], vals, mask=valid)
        return carry

    lax.fori_loop(0, (cnt + 15) >> 4, _scatter, 0)
    pltpu.sync_copy(ob_v, out_hbm.at[pl.ds(base, RPW)])


def kernel(x0, task_ids, out_buf, task, W1, b1, W2, b2):
    ids = task_ids.reshape(N).astype(jnp.int32)
    taskv = jnp.full((16,), task, jnp.int32)
    idxm, counts = _compact(ids, taskv)
    padded = (((counts[:, 0] + GCH - 1) // GCH) * GCH).reshape(16, 2)
    excl = jnp.cumsum(padded, axis=0) - padded
    offs = jnp.broadcast_to(excl.reshape(NW, 1), (NW, 16)).astype(jnp.int32)
    tot2 = padded.sum(axis=0).astype(jnp.int32)
    xg = _gather(x0, idxm, counts, offs)
    hc = _mlp_stream(tot2, xg, W1, b1.reshape(1, H), W2.reshape(1, H),
                     b2.reshape(1))
    out = _scatter_back(idxm, counts, offs, hc, out_buf.reshape(N))
    return out.reshape(N, 1)


# speedup vs baseline: 1.3527x; 1.0058x over previous
import functools

import jax
import jax.numpy as jnp
from jax import lax
from jax.experimental import pallas as pl
from jax.experimental.pallas import tpu as pltpu
from jax.experimental.pallas import tpu_sc as plsc

N = 65536
D = 512
H = 1024

NW = 32
RPW = N // NW
GCH = 64
LOGGCH = 6
HALF = N // 2
BM = 1024
LOGBM = 10
NSTEPS = N // BM

_INV_SQRT2 = 0.7071067811865476

_MESH = plsc.VectorSubcoreMesh(core_axis_name="c", subcore_axis_name="s")


@functools.partial(
    pl.kernel,
    out_type=[
        jax.ShapeDtypeStruct((NW, RPW), jnp.int32),
        jax.ShapeDtypeStruct((NW, 16), jnp.int32),
    ],
    mesh=_MESH,
    compiler_params=pltpu.CompilerParams(needs_layout_passes=False),
    scratch_types=[
        pltpu.VMEM((RPW,), jnp.int32),
        pltpu.VMEM((RPW,), jnp.int32),
        pltpu.VMEM((16,), jnp.int32),
        pltpu.VMEM((16,), jnp.int32),
    ],
)
def _compact(ids_hbm, task_hbm, idx_hbm, cnts_hbm,
             ids_v, idx_v, task_v, cnt_v):
    cid = lax.axis_index("c")
    sid = lax.axis_index("s")
    wid = sid * 2 + cid
    base = wid * RPW
    pltpu.sync_copy(ids_hbm.at[pl.ds(base, RPW)], ids_v)
    pltpu.sync_copy(task_hbm, task_v)
    t = task_v[...]
    lanes = lax.iota(jnp.int32, 16)
    zeros16 = jnp.zeros((16,), jnp.int32)

    def _zero(i, c):
        idx_v[pl.ds(i * 16, 16)] = zeros16
        return c

    lax.fori_loop(0, RPW // 16, _zero, 0)

    def _step(i, ofs):
        v = ids_v[pl.ds(i * 16, 16)]
        m = v == t
        rows = (base + i * 16) + lanes
        cs = plsc.cumsum(jnp.where(m, zeros16 + 1, zeros16))
        pos = ofs + cs - 1
        plsc.store_scatter(idx_v, [pos], rows, mask=m)
        return ofs + jnp.sum(jnp.where(m, zeros16 + 1, zeros16), axis=0)

    ofs = lax.fori_loop(0, RPW // 16, _step, jnp.zeros((16,), jnp.int32))
    cnt_v[...] = ofs
    pltpu.sync_copy(cnt_v, cnts_hbm.at[wid])
    pltpu.sync_copy(idx_v, idx_hbm.at[wid])


@functools.partial(
    pl.kernel,
    out_type=jax.ShapeDtypeStruct((N, D), jnp.float32),
    mesh=_MESH,
    compiler_params=pltpu.CompilerParams(needs_layout_passes=False),
    scratch_types=[
        pltpu.VMEM((RPW,), jnp.int32),
        pltpu.VMEM((16,), jnp.int32),
        pltpu.VMEM((16,), jnp.int32),
        pltpu.VMEM((2, GCH, D), jnp.float32),
        pltpu.SemaphoreType.DMA((2, GCH // 16)),
        pltpu.SemaphoreType.DMA((2,)),
    ],
)
def _gather(x0_hbm, idx_hbm, cnts_hbm, offs_hbm, xg_hbm,
            idx_v, cnt_v, off_v, rows_v, gsem, wsem):
    cid = lax.axis_index("c")
    sid = lax.axis_index("s")
    wid = sid * 2 + cid
    pltpu.sync_copy(idx_hbm.at[wid], idx_v)
    pltpu.sync_copy(cnts_hbm.at[wid], cnt_v)
    pltpu.sync_copy(offs_hbm.at[wid], off_v)
    cnt = jnp.max(cnt_v[...], axis=0)
    off = jnp.max(off_v[...], axis=0)
    nch = (cnt + GCH - 1) >> LOGGCH
    cbase = pl.multiple_of(cid * HALF + off, 8)

    def _gk(c, p, k):
        return pltpu.make_async_copy(
            x0_hbm.at[idx_v.at[pl.ds(c * GCH + k * 16, 16)]],
            rows_v.at[p, pl.ds(k * 16, 16)], gsem.at[p, k])

    def _g_start(c, p):
        for k in range(GCH // 16):
            _gk(c, p, k).start()

    def _g_wait(c, p):
        for k in range(GCH // 16):
            _gk(c, p, k).wait()

    def _w(c, p):
        return pltpu.make_async_copy(
            rows_v.at[p], xg_hbm.at[pl.ds(cbase + c * GCH, GCH)],
            wsem.at[p])

    @pl.when(nch > 0)
    def _():
        _g_start(0, 0)

    def _chunk(c, carry):
        p = c & 1
        _g_wait(c, p)

        @pl.when(c >= 1)
        def _():
            _w(c - 1, 1 - p).wait()
        _w(c, p).start()

        @pl.when(c + 1 < nch)
        def _():
            _g_start(c + 1, 1 - p)
        return carry

    lax.fori_loop(0, nch, _chunk, 0)

    @pl.when(nch > 0)
    def _():
        _w(nch - 1, (nch - 1) & 1).wait()


def _mlp_body(s_ref, w1_ref, b1_ref, w2_ref, b2_ref, x_any, hc_ref,
              xb, sem):
    i = pl.program_id(0)
    na0 = (s_ref[0] + BM - 1) >> LOGBM
    na1 = (s_ref[1] + BM - 1) >> LOGBM
    na = na0 + na1

    def base_of(j):
        return pl.multiple_of(
            jnp.where(j < na0, j << LOGBM, HALF + ((j - na0) << LOGBM)), BM)

    def start(j, p):
        pltpu.make_async_copy(x_any.at[pl.ds(base_of(j), BM), :],
                              xb.at[p], sem.at[p]).start()

    @pl.when(i == 0)
    def _():
        @pl.when(na > 0)
        def _():
            start(0, 0)

        @pl.when(na > 1)
        def _():
            start(1, 1)

        @pl.when(na > 2)
        def _():
            start(2, 2)

    @pl.when(i < na)
    def _():
        @pl.when(i + 3 < na)
        def _():
            start(i + 3, (i + 3) % 4)
        p = i % 4
        pltpu.make_async_copy(x_any.at[pl.ds(base_of(i), BM), :],
                              xb.at[p], sem.at[p]).wait()
        x = xb[p]
        h = jnp.dot(x, w1_ref[...], preferred_element_type=jnp.float32)
        h = h + b1_ref[...]
        g = 0.5 * h * (1.0 + jax.lax.erf(h * _INV_SQRT2))
        o = jnp.sum(g * w2_ref[...], axis=1) + b2_ref[0]
        hc_ref[pl.ds(base_of(i), BM)] = o


def _mlp_stream(tot2, xg, W1, b1r, w2r, b2f):
    grid_spec = pltpu.PrefetchScalarGridSpec(
        num_scalar_prefetch=1,
        grid=(NSTEPS,),
        in_specs=[
            pl.BlockSpec((D, H), lambda i, s: (0, 0)),
            pl.BlockSpec((1, H), lambda i, s: (0, 0)),
            pl.BlockSpec((1, H), lambda i, s: (0, 0)),
            pl.BlockSpec(memory_space=pltpu.SMEM),
            pl.BlockSpec(memory_space=pl.ANY),
        ],
        out_specs=pl.BlockSpec((N,), lambda i, s: (0,)),
        scratch_shapes=[
            pltpu.VMEM((4, BM, D), jnp.float32),
            pltpu.SemaphoreType.DMA((4,)),
        ],
    )
    return pl.pallas_call(
        _mlp_body,
        grid_spec=grid_spec,
        out_shape=jax.ShapeDtypeStruct((N,), jnp.float32),
    )(tot2, W1, b1r, w2r, b2f, xg)


@functools.partial(
    pl.kernel,
    out_type=jax.ShapeDtypeStruct((N,), jnp.float32),
    mesh=_MESH,
    compiler_params=pltpu.CompilerParams(needs_layout_passes=False),
    scratch_types=[
        pltpu.VMEM((RPW,), jnp.int32),
        pltpu.VMEM((RPW,), jnp.float32),
        pltpu.VMEM((RPW,), jnp.float32),
        pltpu.VMEM((16,), jnp.int32),
        pltpu.VMEM((16,), jnp.int32),
    ],
)
def _scatter_back(idx_hbm, cnts_hbm, offs_hbm, hc_hbm, ob_hbm, out_hbm,
                  idx_v, hv_v, ob_v, cnt_v, off_v):
    cid = lax.axis_index("c")
    sid = lax.axis_index("s")
    wid = sid * 2 + cid
    base = wid * RPW
    pltpu.sync_copy(cnts_hbm.at[wid], cnt_v)
    pltpu.sync_copy(offs_hbm.at[wid], off_v)
    pltpu.sync_copy(ob_hbm.at[pl.ds(base, RPW)], ob_v)
    pltpu.sync_copy(idx_hbm.at[wid], idx_v)
    cnt = jnp.max(cnt_v[...], axis=0)
    off = jnp.max(off_v[...], axis=0)
    hoff = pl.multiple_of(cid * HALF + off, 8)
    pltpu.sync_copy(hc_hbm.at[pl.ds(hoff, RPW)], hv_v)
    lanes = lax.iota(jnp.int32, 16)

    def _scatter(j, carry):
        pos = idx_v[pl.ds(j * 16, 16)] - base
        vals = hv_v[pl.ds(j * 16, 16)]
        valid = (j * 16 + lanes) < cnt
        plsc.store_scatter(ob_v, [pos], vals, mask=valid)
        return carry

    lax.fori_loop(0, (cnt + 15) >> 4, _scatter, 0)
    pltpu.sync_copy(ob_v, out_hbm.at[pl.ds(base, RPW)])


def kernel(x0, task_ids, out_buf, task, W1, b1, W2, b2):
    ids = task_ids.reshape(N).astype(jnp.int32)
    taskv = jnp.full((16,), task, jnp.int32)
    idxm, counts = _compact(ids, taskv)
    padded = (((counts[:, 0] + GCH - 1) // GCH) * GCH).reshape(16, 2)
    excl = jnp.cumsum(padded, axis=0) - padded
    offs = jnp.broadcast_to(excl.reshape(NW, 1), (NW, 16)).astype(jnp.int32)
    tot2 = padded.sum(axis=0).astype(jnp.int32)
    xg = _gather(x0, idxm, counts, offs)
    hc = _mlp_stream(tot2, xg, W1, b1.reshape(1, H), W2.reshape(1, H),
                     b2.reshape(1))
    out = _scatter_back(idxm, counts, offs, hc, out_buf.reshape(N))
    return out.reshape(N, 1)
